# baseline (speedup 1.0000x reference)
import functools

import jax
import jax.numpy as jnp
from jax.experimental import pallas as pl
from jax.experimental.pallas import tpu as pltpu

_N = 100000
_BLK = 25000
_TP = 368


def _fused_kernel(x_ref, w1_ref, b1_ref, w2_ref, b2_ref, seq_ref,
                  wihb_ref, wihg_ref, wiho_ref,
                  whhb_ref, whhg_ref, whho_ref,
                  bihb_ref, bihg_ref, biho_ref,
                  bhhb_ref, bhhg_ref, bhho_ref,
                  whb_ref, whg_ref, who_ref,
                  bhb_ref, bhg_ref, bho_ref,
                  beta_ref, gamma_ref, omega_ref,
                  acc_scr, seqp_scr, gi_scr, hs_scr, wseq_scr, wg_scr,
                  whh_scr, whht_scr, bih_scr, whead_scr,
                  *, t_steps, n_blk):
    i = pl.program_id(0)

    h1 = jnp.maximum(
        jnp.dot(x_ref[...].astype(jnp.bfloat16),
                w1_ref[...].astype(jnp.bfloat16),
                preferred_element_type=jnp.float32)
        + b1_ref[...], 0.0)
    h2 = jnp.maximum(
        jnp.dot(h1.astype(jnp.bfloat16), w2_ref[...].astype(jnp.bfloat16),
                preferred_element_type=jnp.float32)
        + b2_ref[...], 0.0)
    part = jnp.sum(h2, axis=0, keepdims=True)

    @pl.when(i == 0)
    def _init():
        acc_scr[...] = part

    @pl.when(i > 0)
    def _acc():
        acc_scr[...] += part

    @pl.when(i == 0)
    def _setup():
        whh_scr[...] = jnp.zeros_like(whh_scr)
        for k in range(3):
            r0 = 128 * k
            s = jnp.float32(0.5 if k < 2 else 1.0)
            wseq_scr[r0:r0 + 64, :] = s * wihb_ref[64 * k:64 * k + 64, 0:8]
            wseq_scr[r0 + 64:r0 + 96, :] = (
                s * wihg_ref[32 * k:32 * k + 32, 0:8])
            wseq_scr[r0 + 96:r0 + 128, :] = (
                s * wiho_ref[32 * k:32 * k + 32, 0:8])
            wg_scr[r0:r0 + 64, :] = s * wihb_ref[64 * k:64 * k + 64, 3:67]
            wg_scr[r0 + 64:r0 + 96, :] = (
                s * wihg_ref[32 * k:32 * k + 32, 3:67])
            wg_scr[r0 + 96:r0 + 128, :] = (
                s * wiho_ref[32 * k:32 * k + 32, 3:67])
            whh_scr[r0:r0 + 64, 0:64] = s * whhb_ref[64 * k:64 * k + 64, :]
            whh_scr[r0 + 64:r0 + 96, 64:96] = (
                s * whhg_ref[32 * k:32 * k + 32, :])
            whh_scr[r0 + 96:r0 + 128, 96:128] = (
                s * whho_ref[32 * k:32 * k + 32, :])
            if k < 2:
                bih_scr[:, r0:r0 + 64] = s * (
                    bihb_ref[:, 64 * k:64 * k + 64]
                    + bhhb_ref[:, 64 * k:64 * k + 64])
                bih_scr[:, r0 + 64:r0 + 96] = s * (
                    bihg_ref[:, 32 * k:32 * k + 32]
                    + bhhg_ref[:, 32 * k:32 * k + 32])
                bih_scr[:, r0 + 96:r0 + 128] = s * (
                    biho_ref[:, 32 * k:32 * k + 32]
                    + bhho_ref[:, 32 * k:32 * k + 32])
            else:
                bih_scr[:, r0:r0 + 64] = bihb_ref[:, 64 * k:64 * k + 64]
                bih_scr[:, r0 + 64:r0 + 96] = bihg_ref[:, 32 * k:32 * k + 32]
                bih_scr[:, r0 + 96:r0 + 128] = (
                    biho_ref[:, 32 * k:32 * k + 32])
        whht_scr[...] = whh_scr[...].T
        whead_scr[...] = jnp.zeros_like(whead_scr)
        whead_scr[0:1, 0:64] = whb_ref[...]
        whead_scr[1:2, 64:96] = whg_ref[...]
        whead_scr[2:3, 96:128] = who_ref[...]

        seqp_scr[...] = jnp.zeros_like(seqp_scr)
        seqp_scr[0:t_steps, 0:3] = seq_ref[...]
        gi_scr[...] = (
            jax.lax.dot_general(seqp_scr[...], wseq_scr[...],
                                (((1,), (1,)), ((), ())),
                                preferred_element_type=jnp.float32)
            + bih_scr[...])
        hs_scr[...] = jnp.zeros_like(hs_scr)

    @pl.when(i == n_blk - 1)
    def _temporal():
        bhhn = jnp.concatenate(
            [bhhb_ref[:, 128:192], bhhg_ref[:, 64:96], bhho_ref[:, 64:96]],
            axis=1)
        g = acc_scr[...] * jnp.float32(1.0 / _N)
        cdims = (((1,), (1,)), ((), ()))
        gi_g = jax.lax.dot_general(g, wg_scr[...], cdims,
                                   preferred_element_type=jnp.float32)
        gi_scr[...] += gi_g

        whht = whht_scr[...].astype(jnp.bfloat16)
        whht_rz = whht[:, 0:256]
        whht_n = whht[:, 256:384]

        def body(t, h):
            gi = gi_scr[pl.ds(t, 1), :]
            hb = h.astype(jnp.bfloat16)
            gh_rz = jnp.dot(hb, whht_rz,
                            preferred_element_type=jnp.float32)
            gh_n = jnp.dot(hb, whht_n,
                           preferred_element_type=jnp.float32)
            r = 0.5 * jnp.tanh(gi[:, 0:128] + gh_rz[:, 0:128]) + 0.5
            u = 0.5 * jnp.tanh(gi[:, 128:256] + gh_rz[:, 128:256]) + 0.5
            n = jnp.tanh(gi[:, 256:384] + r * (gh_n + bhhn))
            h_new = n + u * (h - n)
            hs_scr[pl.ds(t, 1), :] = h_new
            return h_new

        h0 = jnp.zeros((1, 128), jnp.float32)
        jax.lax.fori_loop(0, t_steps, body, h0, unroll=8)

        outt = jax.lax.dot_general(whead_scr[...], hs_scr[...], cdims,
                                   preferred_element_type=jnp.float32)
        beta_ref[...] = jax.nn.softplus(outt[0, 0:t_steps] + bhb_ref[0, 0])
        gamma_ref[...] = (
            jax.nn.softplus(outt[1, 0:t_steps] + bhg_ref[0, 0]) + 1e-6)
        omega_ref[...] = (
            jax.nn.softplus(outt[2, 0:t_steps] + bho_ref[0, 0]) + 1e-6)


def kernel(x, seq_inputs, W1, b1, W2, b2, Wih_b, Whh_b, bih_b, bhh_b, Wh_b,
           bh_b, Wih_g, Whh_g, bih_g, bhh_g, Wh_g, bh_g, Wih_o, Whh_o,
           bih_o, bhh_o, Wh_o, bh_o):
    n, d = x.shape
    h_dim = W1.shape[1]
    t_steps = seq_inputs.shape[1]
    n_blk = n // _BLK

    full = lambda i: (0, 0)
    beta, gamma, omega = pl.pallas_call(
        functools.partial(_fused_kernel, t_steps=t_steps, n_blk=n_blk),
        grid=(n_blk,),
        in_specs=[
            pl.BlockSpec((_BLK, d), lambda i: (i, 0)),
            pl.BlockSpec((d, h_dim), full),
            pl.BlockSpec((1, h_dim), full),
            pl.BlockSpec((h_dim, h_dim), full),
            pl.BlockSpec((1, h_dim), full),
            pl.BlockSpec((t_steps, 3), full),
        ] + [pl.BlockSpec(shape, full) for shape in [
            (192, 67), (96, 67), (96, 67),
            (192, 64), (96, 32), (96, 32),
            (1, 192), (1, 96), (1, 96),
            (1, 192), (1, 96), (1, 96),
            (1, 64), (1, 32), (1, 32),
            (1, 1), (1, 1), (1, 1),
        ]],
        out_specs=[
            pl.BlockSpec((t_steps,), lambda i: (0,)),
            pl.BlockSpec((t_steps,), lambda i: (0,)),
            pl.BlockSpec((t_steps,), lambda i: (0,)),
        ],
        out_shape=[
            jax.ShapeDtypeStruct((t_steps,), jnp.float32),
            jax.ShapeDtypeStruct((t_steps,), jnp.float32),
            jax.ShapeDtypeStruct((t_steps,), jnp.float32),
        ],
        scratch_shapes=[
            pltpu.VMEM((1, 64), jnp.float32),
            pltpu.VMEM((_TP, 8), jnp.float32),
            pltpu.VMEM((_TP, 384), jnp.float32),
            pltpu.VMEM((_TP, 128), jnp.float32),
            pltpu.VMEM((384, 8), jnp.float32),
            pltpu.VMEM((384, 64), jnp.float32),
            pltpu.VMEM((384, 128), jnp.float32),
            pltpu.VMEM((128, 384), jnp.float32),
            pltpu.VMEM((1, 384), jnp.float32),
            pltpu.VMEM((8, 128), jnp.float32),
        ],
        compiler_params=pltpu.CompilerParams(
            dimension_semantics=("arbitrary",)),
    )(x, W1, b1.reshape(1, -1), W2, b2.reshape(1, -1),
      seq_inputs.reshape(t_steps, 3),
      Wih_b, Wih_g, Wih_o,
      Whh_b, Whh_g, Whh_o,
      bih_b.reshape(1, -1), bih_g.reshape(1, -1), bih_o.reshape(1, -1),
      bhh_b.reshape(1, -1), bhh_g.reshape(1, -1), bhh_o.reshape(1, -1),
      Wh_b.reshape(1, -1), Wh_g.reshape(1, -1), Wh_o.reshape(1, -1),
      bh_b.reshape(1, 1), bh_g.reshape(1, 1), bh_o.reshape(1, 1))
    return beta, gamma, omega


# speedup vs baseline: 1.1416x; 1.1416x over previous
import functools

import jax
import jax.numpy as jnp
from jax.experimental import pallas as pl
from jax.experimental.pallas import tpu as pltpu

_N = 100000
_BLK = 20000
_TP = 368


def _fused_kernel(x_ref, w1_ref, b1_ref, w2_ref, b2_ref, seq_ref,
                  wihb_ref, wihg_ref, wiho_ref,
                  whhb_ref, whhg_ref, whho_ref,
                  bihb_ref, bihg_ref, biho_ref,
                  bhhb_ref, bhhg_ref, bhho_ref,
                  whb_ref, whg_ref, who_ref,
                  bhb_ref, bhg_ref, bho_ref,
                  beta_ref, gamma_ref, omega_ref,
                  acc_scr, seqp_scr, gi_scr, hs_scr, wseq_scr, wg_scr,
                  whh_scr, whht_scr, bih_scr, whead_scr,
                  *, t_steps, n_blk):
    i = pl.program_id(0)

    h1 = jnp.maximum(
        jnp.dot(x_ref[...].astype(jnp.bfloat16),
                w1_ref[...].astype(jnp.bfloat16),
                preferred_element_type=jnp.float32)
        + b1_ref[...], 0.0)
    h2 = jnp.maximum(
        jnp.dot(h1.astype(jnp.bfloat16), w2_ref[...].astype(jnp.bfloat16),
                preferred_element_type=jnp.float32)
        + b2_ref[...], 0.0)
    part = jnp.sum(h2, axis=0, keepdims=True)

    @pl.when(i == 0)
    def _init():
        acc_scr[...] = part

    @pl.when(i > 0)
    def _acc():
        acc_scr[...] += part

    @pl.when(i == 0)
    def _setup():
        whh_scr[...] = jnp.zeros_like(whh_scr)
        for k in range(3):
            r0 = 128 * k
            s = jnp.float32(0.5 if k < 2 else 1.0)
            wseq_scr[r0:r0 + 64, :] = s * wihb_ref[64 * k:64 * k + 64, 0:8]
            wseq_scr[r0 + 64:r0 + 96, :] = (
                s * wihg_ref[32 * k:32 * k + 32, 0:8])
            wseq_scr[r0 + 96:r0 + 128, :] = (
                s * wiho_ref[32 * k:32 * k + 32, 0:8])
            wg_scr[r0:r0 + 64, :] = s * wihb_ref[64 * k:64 * k + 64, 3:67]
            wg_scr[r0 + 64:r0 + 96, :] = (
                s * wihg_ref[32 * k:32 * k + 32, 3:67])
            wg_scr[r0 + 96:r0 + 128, :] = (
                s * wiho_ref[32 * k:32 * k + 32, 3:67])
            whh_scr[r0:r0 + 64, 0:64] = s * whhb_ref[64 * k:64 * k + 64, :]
            whh_scr[r0 + 64:r0 + 96, 64:96] = (
                s * whhg_ref[32 * k:32 * k + 32, :])
            whh_scr[r0 + 96:r0 + 128, 96:128] = (
                s * whho_ref[32 * k:32 * k + 32, :])
            if k < 2:
                bih_scr[:, r0:r0 + 64] = s * (
                    bihb_ref[:, 64 * k:64 * k + 64]
                    + bhhb_ref[:, 64 * k:64 * k + 64])
                bih_scr[:, r0 + 64:r0 + 96] = s * (
                    bihg_ref[:, 32 * k:32 * k + 32]
                    + bhhg_ref[:, 32 * k:32 * k + 32])
                bih_scr[:, r0 + 96:r0 + 128] = s * (
                    biho_ref[:, 32 * k:32 * k + 32]
                    + bhho_ref[:, 32 * k:32 * k + 32])
            else:
                bih_scr[:, r0:r0 + 64] = bihb_ref[:, 64 * k:64 * k + 64]
                bih_scr[:, r0 + 64:r0 + 96] = bihg_ref[:, 32 * k:32 * k + 32]
                bih_scr[:, r0 + 96:r0 + 128] = (
                    biho_ref[:, 32 * k:32 * k + 32])
        whht_scr[...] = whh_scr[...].T
        whead_scr[...] = jnp.zeros_like(whead_scr)
        whead_scr[0:1, 0:64] = whb_ref[...]
        whead_scr[1:2, 64:96] = whg_ref[...]
        whead_scr[2:3, 96:128] = who_ref[...]

        seqp_scr[...] = jnp.zeros_like(seqp_scr)
        seqp_scr[0:t_steps, 0:3] = seq_ref[...]
        gi_scr[...] = (
            jax.lax.dot_general(seqp_scr[...], wseq_scr[...],
                                (((1,), (1,)), ((), ())),
                                preferred_element_type=jnp.float32)
            + bih_scr[...])
        hs_scr[...] = jnp.zeros_like(hs_scr)

    @pl.when(i == n_blk - 1)
    def _temporal():
        bhhn = jnp.concatenate(
            [bhhb_ref[:, 128:192], bhhg_ref[:, 64:96], bhho_ref[:, 64:96]],
            axis=1)
        g = acc_scr[...] * jnp.float32(1.0 / _N)
        cdims = (((1,), (1,)), ((), ()))
        gi_g = jax.lax.dot_general(g, wg_scr[...], cdims,
                                   preferred_element_type=jnp.float32)
        gi_scr[...] += gi_g

        whht = whht_scr[...].astype(jnp.bfloat16)
        whht_rz = whht[:, 0:256]
        whht_n = whht[:, 256:384]

        def body(t, h):
            gi = gi_scr[pl.ds(t, 1), :]
            hb = h.astype(jnp.bfloat16)
            gh_rz = jnp.dot(hb, whht_rz,
                            preferred_element_type=jnp.float32)
            gh_n = jnp.dot(hb, whht_n,
                           preferred_element_type=jnp.float32)
            r = 0.5 * jnp.tanh(gi[:, 0:128] + gh_rz[:, 0:128]) + 0.5
            u = 0.5 * jnp.tanh(gi[:, 128:256] + gh_rz[:, 128:256]) + 0.5
            n = jnp.tanh(gi[:, 256:384] + r * (gh_n + bhhn))
            h_new = n + u * (h - n)
            hs_scr[pl.ds(t, 1), :] = h_new
            return h_new

        h0 = jnp.zeros((1, 128), jnp.float32)
        jax.lax.fori_loop(0, t_steps, body, h0, unroll=16)

        outt = jax.lax.dot_general(whead_scr[...], hs_scr[...], cdims,
                                   preferred_element_type=jnp.float32)
        beta_ref[...] = jax.nn.softplus(outt[0, 0:t_steps] + bhb_ref[0, 0])
        gamma_ref[...] = (
            jax.nn.softplus(outt[1, 0:t_steps] + bhg_ref[0, 0]) + 1e-6)
        omega_ref[...] = (
            jax.nn.softplus(outt[2, 0:t_steps] + bho_ref[0, 0]) + 1e-6)


def kernel(x, seq_inputs, W1, b1, W2, b2, Wih_b, Whh_b, bih_b, bhh_b, Wh_b,
           bh_b, Wih_g, Whh_g, bih_g, bhh_g, Wh_g, bh_g, Wih_o, Whh_o,
           bih_o, bhh_o, Wh_o, bh_o):
    n, d = x.shape
    h_dim = W1.shape[1]
    t_steps = seq_inputs.shape[1]
    n_blk = n // _BLK

    full = lambda i: (0, 0)
    beta, gamma, omega = pl.pallas_call(
        functools.partial(_fused_kernel, t_steps=t_steps, n_blk=n_blk),
        grid=(n_blk,),
        in_specs=[
            pl.BlockSpec((_BLK, d), lambda i: (i, 0)),
            pl.BlockSpec((d, h_dim), full),
            pl.BlockSpec((1, h_dim), full),
            pl.BlockSpec((h_dim, h_dim), full),
            pl.BlockSpec((1, h_dim), full),
            pl.BlockSpec((t_steps, 3), full),
        ] + [pl.BlockSpec(shape, full) for shape in [
            (192, 67), (96, 67), (96, 67),
            (192, 64), (96, 32), (96, 32),
            (1, 192), (1, 96), (1, 96),
            (1, 192), (1, 96), (1, 96),
            (1, 64), (1, 32), (1, 32),
            (1, 1), (1, 1), (1, 1),
        ]],
        out_specs=[
            pl.BlockSpec((t_steps,), lambda i: (0,)),
            pl.BlockSpec((t_steps,), lambda i: (0,)),
            pl.BlockSpec((t_steps,), lambda i: (0,)),
        ],
        out_shape=[
            jax.ShapeDtypeStruct((t_steps,), jnp.float32),
            jax.ShapeDtypeStruct((t_steps,), jnp.float32),
            jax.ShapeDtypeStruct((t_steps,), jnp.float32),
        ],
        scratch_shapes=[
            pltpu.VMEM((1, 64), jnp.float32),
            pltpu.VMEM((_TP, 8), jnp.float32),
            pltpu.VMEM((_TP, 384), jnp.float32),
            pltpu.VMEM((_TP, 128), jnp.float32),
            pltpu.VMEM((384, 8), jnp.float32),
            pltpu.VMEM((384, 64), jnp.float32),
            pltpu.VMEM((384, 128), jnp.float32),
            pltpu.VMEM((128, 384), jnp.float32),
            pltpu.VMEM((1, 384), jnp.float32),
            pltpu.VMEM((8, 128), jnp.float32),
        ],
        compiler_params=pltpu.CompilerParams(
            dimension_semantics=("arbitrary",)),
    )(x, W1, b1.reshape(1, -1), W2, b2.reshape(1, -1),
      seq_inputs.reshape(t_steps, 3),
      Wih_b, Wih_g, Wih_o,
      Whh_b, Whh_g, Whh_o,
      bih_b.reshape(1, -1), bih_g.reshape(1, -1), bih_o.reshape(1, -1),
      bhh_b.reshape(1, -1), bhh_g.reshape(1, -1), bhh_o.reshape(1, -1),
      Wh_b.reshape(1, -1), Wh_g.reshape(1, -1), Wh_o.reshape(1, -1),
      bh_b.reshape(1, 1), bh_g.reshape(1, 1), bh_o.reshape(1, 1))
    return beta, gamma, omega


# speedup vs baseline: 1.1464x; 1.0042x over previous
import functools

import jax
import jax.numpy as jnp
from jax.experimental import pallas as pl
from jax.experimental.pallas import tpu as pltpu

_N = 100000
_BLK = 20000
_TP = 368


def _fused_kernel(x_ref, w1_ref, b1_ref, w2_ref, b2_ref, seq_ref,
                  wihb_ref, wihg_ref, wiho_ref,
                  whhb_ref, whhg_ref, whho_ref,
                  bihb_ref, bihg_ref, biho_ref,
                  bhhb_ref, bhhg_ref, bhho_ref,
                  whb_ref, whg_ref, who_ref,
                  bhb_ref, bhg_ref, bho_ref,
                  beta_ref, gamma_ref, omega_ref,
                  acc_scr, seqp_scr, gi_scr, hs_scr, wseq_scr, wg_scr,
                  whh_scr, whht_scr, bih_scr, whead_scr,
                  *, t_steps, n_blk):
    i = pl.program_id(0)

    h1 = jnp.maximum(
        jnp.dot(x_ref[...].astype(jnp.bfloat16),
                w1_ref[...].astype(jnp.bfloat16),
                preferred_element_type=jnp.float32)
        + b1_ref[...], 0.0)
    h2 = jnp.maximum(
        jnp.dot(h1.astype(jnp.bfloat16), w2_ref[...].astype(jnp.bfloat16),
                preferred_element_type=jnp.float32)
        + b2_ref[...], 0.0)
    part = jnp.sum(h2, axis=0, keepdims=True)

    @pl.when(i == 0)
    def _init():
        acc_scr[...] = part

    @pl.when(i > 0)
    def _acc():
        acc_scr[...] += part

    @pl.when(i == 0)
    def _setup():
        whh_scr[...] = jnp.zeros_like(whh_scr)
        for k in range(3):
            r0 = 128 * k
            s = jnp.float32(0.5 if k < 2 else 1.0)
            wseq_scr[r0:r0 + 64, :] = s * wihb_ref[64 * k:64 * k + 64, 0:8]
            wseq_scr[r0 + 64:r0 + 96, :] = (
                s * wihg_ref[32 * k:32 * k + 32, 0:8])
            wseq_scr[r0 + 96:r0 + 128, :] = (
                s * wiho_ref[32 * k:32 * k + 32, 0:8])
            wg_scr[r0:r0 + 64, :] = s * wihb_ref[64 * k:64 * k + 64, 3:67]
            wg_scr[r0 + 64:r0 + 96, :] = (
                s * wihg_ref[32 * k:32 * k + 32, 3:67])
            wg_scr[r0 + 96:r0 + 128, :] = (
                s * wiho_ref[32 * k:32 * k + 32, 3:67])
            whh_scr[r0:r0 + 64, 0:64] = s * whhb_ref[64 * k:64 * k + 64, :]
            whh_scr[r0 + 64:r0 + 96, 64:96] = (
                s * whhg_ref[32 * k:32 * k + 32, :])
            whh_scr[r0 + 96:r0 + 128, 96:128] = (
                s * whho_ref[32 * k:32 * k + 32, :])
            if k < 2:
                bih_scr[:, r0:r0 + 64] = s * (
                    bihb_ref[:, 64 * k:64 * k + 64]
                    + bhhb_ref[:, 64 * k:64 * k + 64])
                bih_scr[:, r0 + 64:r0 + 96] = s * (
                    bihg_ref[:, 32 * k:32 * k + 32]
                    + bhhg_ref[:, 32 * k:32 * k + 32])
                bih_scr[:, r0 + 96:r0 + 128] = s * (
                    biho_ref[:, 32 * k:32 * k + 32]
                    + bhho_ref[:, 32 * k:32 * k + 32])
            else:
                bih_scr[:, r0:r0 + 64] = bihb_ref[:, 64 * k:64 * k + 64]
                bih_scr[:, r0 + 64:r0 + 96] = bihg_ref[:, 32 * k:32 * k + 32]
                bih_scr[:, r0 + 96:r0 + 128] = (
                    biho_ref[:, 32 * k:32 * k + 32])
        whht_scr[...] = whh_scr[...].T
        whead_scr[...] = jnp.zeros_like(whead_scr)
        whead_scr[0:1, 0:64] = whb_ref[...]
        whead_scr[1:2, 64:96] = whg_ref[...]
        whead_scr[2:3, 96:128] = who_ref[...]

        seqp_scr[...] = jnp.zeros_like(seqp_scr)
        seqp_scr[0:t_steps, 0:3] = seq_ref[...]
        gi_scr[...] = (
            jax.lax.dot_general(seqp_scr[...], wseq_scr[...],
                                (((1,), (1,)), ((), ())),
                                preferred_element_type=jnp.float32)
            + bih_scr[...])
        hs_scr[...] = jnp.zeros_like(hs_scr)

    @pl.when(i == n_blk - 1)
    def _temporal():
        bhhn = jnp.concatenate(
            [bhhb_ref[:, 128:192], bhhg_ref[:, 64:96], bhho_ref[:, 64:96]],
            axis=1)
        g = acc_scr[...] * jnp.float32(1.0 / _N)
        cdims = (((1,), (1,)), ((), ()))
        gi_g = jax.lax.dot_general(g, wg_scr[...], cdims,
                                   preferred_element_type=jnp.float32)
        gi_scr[...] += gi_g

        whht = whht_scr[...].astype(jnp.bfloat16)
        whht_rz = whht[:, 0:256]
        whht_n = whht[:, 256:384]

        def body(t, h):
            gi = gi_scr[pl.ds(t, 1), :]
            hb = h.astype(jnp.bfloat16)
            gh_rz = jnp.dot(hb, whht_rz,
                            preferred_element_type=jnp.float32)
            gh_n = jnp.dot(hb, whht_n,
                           preferred_element_type=jnp.float32)
            r = 0.5 * jnp.tanh(gi[:, 0:128] + gh_rz[:, 0:128]) + 0.5
            u = 0.5 * jnp.tanh(gi[:, 128:256] + gh_rz[:, 128:256]) + 0.5
            n = jnp.tanh(gi[:, 256:384] + r * (gh_n + bhhn))
            h_new = n + u * (h - n)
            hs_scr[pl.ds(t, 1), :] = h_new
            return h_new

        h0 = jnp.zeros((1, 128), jnp.float32)
        jax.lax.fori_loop(0, t_steps, body, h0, unroll=32)

        outt = jax.lax.dot_general(whead_scr[...], hs_scr[...], cdims,
                                   preferred_element_type=jnp.float32)
        beta_ref[...] = jax.nn.softplus(outt[0, 0:t_steps] + bhb_ref[0, 0])
        gamma_ref[...] = (
            jax.nn.softplus(outt[1, 0:t_steps] + bhg_ref[0, 0]) + 1e-6)
        omega_ref[...] = (
            jax.nn.softplus(outt[2, 0:t_steps] + bho_ref[0, 0]) + 1e-6)


def kernel(x, seq_inputs, W1, b1, W2, b2, Wih_b, Whh_b, bih_b, bhh_b, Wh_b,
           bh_b, Wih_g, Whh_g, bih_g, bhh_g, Wh_g, bh_g, Wih_o, Whh_o,
           bih_o, bhh_o, Wh_o, bh_o):
    n, d = x.shape
    h_dim = W1.shape[1]
    t_steps = seq_inputs.shape[1]
    n_blk = n // _BLK

    full = lambda i: (0, 0)
    beta, gamma, omega = pl.pallas_call(
        functools.partial(_fused_kernel, t_steps=t_steps, n_blk=n_blk),
        grid=(n_blk,),
        in_specs=[
            pl.BlockSpec((_BLK, d), lambda i: (i, 0)),
            pl.BlockSpec((d, h_dim), full),
            pl.BlockSpec((1, h_dim), full),
            pl.BlockSpec((h_dim, h_dim), full),
            pl.BlockSpec((1, h_dim), full),
            pl.BlockSpec((t_steps, 3), full),
        ] + [pl.BlockSpec(shape, full) for shape in [
            (192, 67), (96, 67), (96, 67),
            (192, 64), (96, 32), (96, 32),
            (1, 192), (1, 96), (1, 96),
            (1, 192), (1, 96), (1, 96),
            (1, 64), (1, 32), (1, 32),
            (1, 1), (1, 1), (1, 1),
        ]],
        out_specs=[
            pl.BlockSpec((t_steps,), lambda i: (0,)),
            pl.BlockSpec((t_steps,), lambda i: (0,)),
            pl.BlockSpec((t_steps,), lambda i: (0,)),
        ],
        out_shape=[
            jax.ShapeDtypeStruct((t_steps,), jnp.float32),
            jax.ShapeDtypeStruct((t_steps,), jnp.float32),
            jax.ShapeDtypeStruct((t_steps,), jnp.float32),
        ],
        scratch_shapes=[
            pltpu.VMEM((1, 64), jnp.float32),
            pltpu.VMEM((_TP, 8), jnp.float32),
            pltpu.VMEM((_TP, 384), jnp.float32),
            pltpu.VMEM((_TP, 128), jnp.float32),
            pltpu.VMEM((384, 8), jnp.float32),
            pltpu.VMEM((384, 64), jnp.float32),
            pltpu.VMEM((384, 128), jnp.float32),
            pltpu.VMEM((128, 384), jnp.float32),
            pltpu.VMEM((1, 384), jnp.float32),
            pltpu.VMEM((8, 128), jnp.float32),
        ],
        compiler_params=pltpu.CompilerParams(
            dimension_semantics=("arbitrary",)),
    )(x, W1, b1.reshape(1, -1), W2, b2.reshape(1, -1),
      seq_inputs.reshape(t_steps, 3),
      Wih_b, Wih_g, Wih_o,
      Whh_b, Whh_g, Whh_o,
      bih_b.reshape(1, -1), bih_g.reshape(1, -1), bih_o.reshape(1, -1),
      bhh_b.reshape(1, -1), bhh_g.reshape(1, -1), bhh_o.reshape(1, -1),
      Wh_b.reshape(1, -1), Wh_g.reshape(1, -1), Wh_o.reshape(1, -1),
      bh_b.reshape(1, 1), bh_g.reshape(1, 1), bh_o.reshape(1, 1))
    return beta, gamma, omega
